# baseline (speedup 1.0000x reference)
import functools

import jax
import jax.numpy as jnp
from jax import lax
from jax.experimental import pallas as pl
from jax.experimental.pallas import tpu as pltpu
from jax.experimental.pallas import tpu_sc as plsc

_EMBED = 300
_NC = 2
_NS = 16
_NW = _NC * _NS


@functools.partial(jax.jit, static_argnums=(2, 3))
def _gather(table, idx, n_per_w, chunk):
    n_total = idx.shape[0]
    nchunks = n_per_w // chunk
    mesh = plsc.VectorSubcoreMesh(core_axis_name="c", subcore_axis_name="s")

    @functools.partial(
        pl.kernel,
        mesh=mesh,
        out_type=jax.ShapeDtypeStruct((n_total, _EMBED), jnp.float32),
        scratch_types=[
            pltpu.VMEM((n_per_w,), jnp.int32),
            pltpu.VMEM((chunk, _EMBED), jnp.float32),
            pltpu.SemaphoreType.DMA,
        ],
        compiler_params=pltpu.CompilerParams(use_tc_tiling_on_sc=False),
    )
    def k(table_hbm, idx_hbm, out_hbm, idx_v, rows_v, gsem):
        wid = lax.axis_index("s") * _NC + lax.axis_index("c")
        base = wid * n_per_w
        pltpu.sync_copy(idx_hbm.at[pl.ds(base, n_per_w)], idx_v)

        def body(c, carry):
            off = c * chunk
            pltpu.async_copy(
                table_hbm.at[idx_v.at[pl.ds(off, chunk)]], rows_v, gsem
            ).wait()
            pltpu.sync_copy(rows_v, out_hbm.at[pl.ds(base + off, chunk)])
            return carry

        lax.fori_loop(0, nchunks, body, 0)

    return k(table, idx)


def kernel(sequence, segment_label, token_table):
    B, L = sequence.shape
    n_total = B * L
    seq = sequence.reshape(n_total).astype(jnp.int32)
    out = _gather(token_table, seq, n_total // _NW, 128)
    return out.reshape(B, L, _EMBED)


# speedup vs baseline: 1.6619x; 1.6619x over previous
import functools

import jax
import jax.numpy as jnp
from jax import lax
from jax.experimental import pallas as pl
from jax.experimental.pallas import tpu as pltpu
from jax.experimental.pallas import tpu_sc as plsc

_EMBED = 300
_DPAD = 384
_NC = 2
_NS = 16
_NW = _NC * _NS


@functools.partial(jax.jit, static_argnums=(2, 3))
def _gather(table_pad, idx, n_per_w, chunk):
    n_total = idx.shape[0]
    ngroups = n_per_w // chunk // 2
    mesh = plsc.VectorSubcoreMesh(core_axis_name="c", subcore_axis_name="s")

    @functools.partial(
        pl.kernel,
        mesh=mesh,
        out_type=jax.ShapeDtypeStruct((n_total, _DPAD), jnp.float32),
        scratch_types=[
            pltpu.VMEM((n_per_w,), jnp.int32),
            pltpu.VMEM((chunk, _DPAD), jnp.float32),
            pltpu.VMEM((chunk, _DPAD), jnp.float32),
            pltpu.SemaphoreType.DMA,
            pltpu.SemaphoreType.DMA,
            pltpu.SemaphoreType.DMA,
            pltpu.SemaphoreType.DMA,
        ],
    )
    def k(table_hbm, idx_hbm, out_hbm, idx_v, buf0, buf1, g0, g1, w0, w1):
        wid = lax.axis_index("s") * _NC + lax.axis_index("c")
        base = wid * n_per_w
        pltpu.sync_copy(idx_hbm.at[pl.ds(base, n_per_w)], idx_v)
        bufs, gs, ws = (buf0, buf1), (g0, g1), (w0, w1)

        def gstart(c, b):
            pltpu.async_copy(
                table_hbm.at[idx_v.at[pl.ds(c * chunk, chunk)]], bufs[b], gs[b]
            )

        def gwait(b):
            pltpu.make_async_copy(table_hbm.at[pl.ds(0, chunk)], bufs[b], gs[b]).wait()

        def wstart(c, b):
            pltpu.async_copy(
                bufs[b], out_hbm.at[pl.ds(base + c * chunk, chunk)], ws[b]
            )

        def wwait(b):
            pltpu.make_async_copy(bufs[b], out_hbm.at[pl.ds(base, chunk)], ws[b]).wait()

        gstart(0, 0)

        def body(g, carry):
            c0 = 2 * g
            gwait(0)

            @pl.when(g >= 1)
            def _():
                wwait(1)

            gstart(c0 + 1, 1)
            wstart(c0, 0)
            gwait(1)
            wwait(0)

            @pl.when(g + 1 < ngroups)
            def _():
                gstart(c0 + 2, 0)

            wstart(c0 + 1, 1)
            return carry

        lax.fori_loop(0, ngroups, body, 0)
        wwait(1)

    return k(table_pad, idx)


def kernel(sequence, segment_label, token_table):
    B, L = sequence.shape
    n_total = B * L
    seq = sequence.reshape(n_total).astype(jnp.int32)
    table_pad = jnp.pad(token_table, ((0, 0), (0, _DPAD - _EMBED)))
    out_pad = _gather(table_pad, seq, n_total // _NW, 128)
    return out_pad[:, :_EMBED].reshape(B, L, _EMBED)
